# baseline (speedup 1.0000x reference)
import functools

import jax
import jax.numpy as jnp
from jax import lax
from jax.experimental import pallas as pl
from jax.experimental.pallas import tpu as pltpu
from jax.experimental.pallas import tpu_sc as plsc

_INFO = plsc.get_sparse_core_info()
_NC = _INFO.num_cores
_NS = _INFO.num_subcores
_NW = _NC * _NS

_B = 16384
_D = 64
_V = 1000000
_R = 1000
_NQ = 7812
_QTAIL = _NQ * 128
_QPER = _NQ // _NW
_QREM = _NQ % _NW
_F = 16
_SR = _B + _NW * _F

_MESH = plsc.VectorSubcoreMesh(core_axis_name="c", subcore_axis_name="s")


@functools.partial(
    pl.kernel,
    mesh=_MESH,
    out_type=jax.ShapeDtypeStruct((_SR, 128), jnp.float32),
    scratch_types=[
        pltpu.VMEM((_B + 16,), jnp.int32),
        pltpu.VMEM((_B + 16,), jnp.int32),
        pltpu.VMEM((_B + 16,), jnp.int32),
        pltpu.VMEM((288,), jnp.int32),
        pltpu.VMEM((288,), jnp.int32),
        pltpu.VMEM((288,), jnp.int32),
        pltpu.VMEM((288,), jnp.int32),
        pltpu.VMEM((6, _D, 128), jnp.float32),
        pltpu.VMEM((_F, 128), jnp.float32),
        pltpu.VMEM((16,), jnp.int32),
        pltpu.SMEM((8,), jnp.int32),
        pltpu.SemaphoreType.DMA,
        pltpu.SemaphoreType.DMA,
        pltpu.SemaphoreType.DMA,
        pltpu.SemaphoreType.DMA,
        pltpu.SemaphoreType.DMA,
        pltpu.SemaphoreType.DMA,
        pltpu.SemaphoreType.DMA,
    ],
    compiler_params=pltpu.CompilerParams(
        use_tc_tiling_on_sc=True, needs_layout_passes=False
    ),
)
def _ent_gather(tabT, e_hbm, out, ev, lst_p, srt_p, hist, st0, cur,
                nbl, blk, fbuf, fpos, cnt,
                sem0, sem1, sem2, sem3, sem4, sem5, semf):
    wid = lax.axis_index("s") * _NC + lax.axis_index("c")
    qlo = wid * _QPER + jnp.minimum(wid, _QREM)
    nq = jnp.where(wid < _QREM, _QPER + 1, _QPER)
    lane = lax.iota(jnp.int32, 16)
    zeros16 = jnp.zeros((16,), jnp.int32)

    pltpu.sync_copy(e_hbm, ev.at[pl.ds(0, _B)])

    cnt[0] = 0

    def filt(i, _):
        x = ev[pl.ds(i * 16, 16)]
        q = lax.shift_right_logical(x, 7)
        m = (q >= qlo) & (q < qlo + nq)
        n = plsc.all_reduce_population_count(m)[0]

        @pl.when(n > 0)
        def _():
            offs = plsc.cumsum(jnp.where(m, 1, 0)) - 1 + cnt[0]
            plsc.store_scatter(lst_p, [offs], lane + i * 16, mask=m)

        cnt[0] = cnt[0] + n
        return 0

    lax.fori_loop(0, _B // 16, filt, 0)
    lcount = cnt[0]

    def zero18(i, _):
        hist[pl.ds(i * 16, 16)] = zeros16
        return 0

    lax.fori_loop(0, 288 // 16, zero18, 0)

    def count1(j, _):
        p_j = lst_p[pl.ds(j, 16)][0]
        e_j = ev[pl.ds(p_j, 16)][0]
        k = lax.shift_right_logical(e_j, 7) - qlo
        w = hist[pl.ds(k, 16)]
        hist[pl.ds(k, 16)] = jnp.where(lane == 0, w + 1, w)
        return 0

    lax.fori_loop(0, lcount, count1, 0)

    cnt[1] = 0

    def scan1(i, _):
        v = hist[pl.ds(i * 16, 16)]
        c = plsc.cumsum(v)
        run = cnt[1]
        st0[pl.ds(i * 16, 16)] = run + c - v
        cur[pl.ds(i * 16, 16)] = run + c - v
        cnt[1] = run + c[15]
        return 0

    lax.fori_loop(0, 288 // 16, scan1, 0)

    def place(j, _):
        p_j = lst_p[pl.ds(j, 16)][0]
        e_j = ev[pl.ds(p_j, 16)][0]
        k = lax.shift_right_logical(e_j, 7) - qlo
        o = cur[pl.ds(k, 16)][0]
        w2 = srt_p[pl.ds(o, 16)]
        srt_p[pl.ds(o, 16)] = jnp.where(
            lane == 0, lax.broadcast(p_j, (16,)), w2
        )
        wc = cur[pl.ds(k, 16)]
        cur[pl.ds(k, 16)] = jnp.where(lane == 0, wc + 1, wc)
        return 0

    lax.fori_loop(0, lcount, place, 0)

    def zeronbl(i, _):
        nbl[pl.ds(i * 16, 16)] = zeros16
        return 0

    lax.fori_loop(0, 288 // 16, zeronbl, 0)
    cnt[2] = 0

    def compact(i, _):
        kv = lane + i * 16
        cv = hist[pl.ds(i * 16, 16)]
        m = (cv > 0) & (kv < nq)
        nc = cnt[2]
        offs = plsc.cumsum(jnp.where(m, 1, 0)) - 1 + nc
        plsc.store_scatter(nbl, [offs], kv, mask=m)
        cnt[2] = nc + plsc.all_reduce_population_count(m)[0]
        return 0

    lax.fori_loop(0, 256 // 16, compact, 0)
    nn = cnt[2]

    cnt[3] = 0

    def append_row(j, bref):
        pos = srt_p[pl.ds(j, 16)][0]
        ipos = ev[pl.ds(pos, 16)][0] & 127
        fc = cnt[3]
        for k2 in range(_D // 16):
            g = plsc.load_gather(
                bref, [lane + k2 * 16, lax.broadcast(ipos, (16,))]
            )
            fbuf[fc, pl.ds(k2 * 16, 16)] = g
        fpos[...] = jnp.where(
            lane == fc, lax.broadcast(pos, (16,)), fpos[...]
        )
        fc = fc + 1

        @pl.when(fc == _F)
        def _():
            pltpu.async_copy(fbuf, out.at[fpos], semf).wait()

        cnt[3] = jnp.where(fc == _F, 0, fc)
        return 0

    def process_block(k, bref):
        lo = st0[pl.ds(k, 16)][0]
        c = hist[pl.ds(k, 16)][0]
        lax.fori_loop(lo, lo + c, lambda j, _: append_row(j, bref), 0)

    sems = (sem0, sem1, sem2, sem3, sem4, sem5)

    def blkid(idx):
        return nbl[pl.ds(jnp.minimum(idx, 271), 16)][0]

    def issue(idx, b, sem):
        q = qlo + blkid(idx)
        pltpu.async_copy(tabT.at[:, pl.ds(q * 128, 128)], blk.at[b], sem)

    def wait(b, sem):
        pltpu.make_async_copy(
            tabT.at[:, pl.ds(0, 128)], blk.at[b], sem
        ).wait()

    for b in range(6):
        issue(b, b, sems[b])

    def hexa(t, _):
        for b in range(6):
            idx = t * 6 + b
            wait(b, sems[b])
            process_block(blkid(idx), blk.at[b])
            issue(idx + 6, b, sems[b])
        return 0

    nhex = lax.div(nn, 6) + 1
    lax.fori_loop(0, nhex, hexa, 0)
    for b in range(6):
        wait(b, sems[b])

    fc = cnt[3]
    dump = _B + wid * _F + lane
    fpos[...] = jnp.where(lane < fc, fpos[...], dump)
    pltpu.async_copy(fbuf, out.at[fpos], semf).wait()


def _tc_body(relT_ref, tailT_ref, r_ref, e_ref, rel_out_ref, tail_out_ref):
    r = r_ref[0, 0, :]
    e = e_ref[0, 0, :]
    iota_r = lax.broadcasted_iota(jnp.int32, (_R, 512), 0)
    oh_r = (iota_r == r[None, :]).astype(jnp.float32)
    rel_out_ref[...] = jnp.dot(
        relT_ref[...], oh_r, preferred_element_type=jnp.float32,
        precision=lax.Precision.HIGHEST
    )
    iota_t = lax.broadcasted_iota(jnp.int32, (_D, 512), 0)
    oh_t = (iota_t == (e[None, :] - _QTAIL)).astype(jnp.float32)
    tail_out_ref[...] = jnp.dot(
        tailT_ref[...], oh_t, preferred_element_type=jnp.float32,
        precision=lax.Precision.HIGHEST
    )


_tc_call = pl.pallas_call(
    _tc_body,
    grid=(_B // 512,),
    in_specs=[
        pl.BlockSpec((_D, _R), lambda j: (0, 0)),
        pl.BlockSpec((_D, _D), lambda j: (0, 0)),
        pl.BlockSpec((1, 1, 512), lambda j: (j, 0, 0)),
        pl.BlockSpec((1, 1, 512), lambda j: (j, 0, 0)),
    ],
    out_specs=[
        pl.BlockSpec((_D, 512), lambda j: (0, j)),
        pl.BlockSpec((_D, 512), lambda j: (0, j)),
    ],
    out_shape=[
        jax.ShapeDtypeStruct((_D, _B), jnp.float32),
        jax.ShapeDtypeStruct((_D, _B), jnp.float32),
    ],
)


def kernel(entity_table, relation_table, e, r):
    e = e.astype(jnp.int32)
    r = r.astype(jnp.int32)
    tabT = entity_table.T
    relT = relation_table.T
    tailT = lax.slice(tabT, (0, _QTAIL), (_D, _V))

    scratch = _ent_gather(tabT, e)
    relT_out, tailT_out = _tc_call(
        relT, tailT, r.reshape(_B // 512, 1, 512), e.reshape(_B // 512, 1, 512)
    )

    ent_sc = scratch[:_B, :_D]
    ent_emb = jnp.where((e >= _QTAIL)[:, None], tailT_out.T, ent_sc)
    rel_emb = relT_out.T
    return ent_emb, rel_emb


# speedup vs baseline: 1.0665x; 1.0665x over previous
import functools

import jax
import jax.numpy as jnp
from jax import lax
from jax.experimental import pallas as pl
from jax.experimental.pallas import tpu as pltpu
from jax.experimental.pallas import tpu_sc as plsc

_INFO = plsc.get_sparse_core_info()
_NC = _INFO.num_cores
_NS = _INFO.num_subcores
_NW = _NC * _NS

_B = 16384
_D = 64
_V = 1000000
_R = 1000
_NQ = 7812
_QTAIL = _NQ * 128
_QPER = _NQ // _NW
_QREM = _NQ % _NW
_F = 16
_SR = _B + _NW * _F

_MESH = plsc.VectorSubcoreMesh(core_axis_name="c", subcore_axis_name="s")


@functools.partial(
    pl.kernel,
    mesh=_MESH,
    out_type=jax.ShapeDtypeStruct((_SR, 128), jnp.float32),
    scratch_types=[
        pltpu.VMEM((_B + 16,), jnp.int32),
        pltpu.VMEM((_B + 16,), jnp.int32),
        pltpu.VMEM((_B + 16,), jnp.int32),
        pltpu.VMEM((288,), jnp.int32),
        pltpu.VMEM((288,), jnp.int32),
        pltpu.VMEM((288,), jnp.int32),
        pltpu.VMEM((288,), jnp.int32),
        pltpu.VMEM((6, _D, 128), jnp.float32),
        pltpu.VMEM((_F, 128), jnp.float32),
        pltpu.VMEM((16,), jnp.int32),
        pltpu.SMEM((8,), jnp.int32),
        pltpu.SemaphoreType.DMA,
        pltpu.SemaphoreType.DMA,
        pltpu.SemaphoreType.DMA,
        pltpu.SemaphoreType.DMA,
        pltpu.SemaphoreType.DMA,
        pltpu.SemaphoreType.DMA,
        pltpu.SemaphoreType.DMA,
    ],
    compiler_params=pltpu.CompilerParams(
        use_tc_tiling_on_sc=True, needs_layout_passes=False
    ),
)
def _ent_gather(tabT, e_hbm, out, ev, lst_p, srt_p, hist, st0, cur,
                nbl, blk, fbuf, fpos, cnt,
                sem0, sem1, sem2, sem3, sem4, sem5, semf):
    wid = lax.axis_index("s") * _NC + lax.axis_index("c")
    qlo = wid * _QPER + jnp.minimum(wid, _QREM)
    nq = jnp.where(wid < _QREM, _QPER + 1, _QPER)
    lane = lax.iota(jnp.int32, 16)
    zeros16 = jnp.zeros((16,), jnp.int32)

    pltpu.sync_copy(e_hbm, ev.at[pl.ds(0, _B)])

    cnt[0] = 0

    def filt(i, _):
        x = ev[pl.ds(i * 16, 16)]
        q = lax.shift_right_logical(x, 7)
        m = (q >= qlo) & (q < qlo + nq)
        c = plsc.cumsum(jnp.where(m, 1, 0))
        offs = c - 1 + cnt[0]
        plsc.store_scatter(lst_p, [offs], lane + i * 16, mask=m)
        cnt[0] = cnt[0] + c[15]
        return 0

    lax.fori_loop(0, _B // 16, filt, 0)
    lcount = cnt[0]

    def zero18(i, _):
        hist[pl.ds(i * 16, 16)] = zeros16
        return 0

    lax.fori_loop(0, 288 // 16, zero18, 0)

    def count1(j, _):
        p_j = lst_p[pl.ds(j, 16)][0]
        e_j = ev[pl.ds(p_j, 16)][0]
        k = lax.shift_right_logical(e_j, 7) - qlo
        w = hist[pl.ds(k, 16)]
        hist[pl.ds(k, 16)] = jnp.where(lane == 0, w + 1, w)
        return 0

    lax.fori_loop(0, lcount, count1, 0)

    cnt[1] = 0

    def scan1(i, _):
        v = hist[pl.ds(i * 16, 16)]
        c = plsc.cumsum(v)
        run = cnt[1]
        st0[pl.ds(i * 16, 16)] = run + c - v
        cur[pl.ds(i * 16, 16)] = run + c - v
        cnt[1] = run + c[15]
        return 0

    lax.fori_loop(0, 288 // 16, scan1, 0)

    def place(j, _):
        p_j = lst_p[pl.ds(j, 16)][0]
        e_j = ev[pl.ds(p_j, 16)][0]
        k = lax.shift_right_logical(e_j, 7) - qlo
        o = cur[pl.ds(k, 16)][0]
        w2 = srt_p[pl.ds(o, 16)]
        srt_p[pl.ds(o, 16)] = jnp.where(
            lane == 0, lax.broadcast(p_j, (16,)), w2
        )
        wc = cur[pl.ds(k, 16)]
        cur[pl.ds(k, 16)] = jnp.where(lane == 0, wc + 1, wc)
        return 0

    lax.fori_loop(0, lcount, place, 0)

    def zeronbl(i, _):
        nbl[pl.ds(i * 16, 16)] = zeros16
        return 0

    lax.fori_loop(0, 288 // 16, zeronbl, 0)
    cnt[2] = 0

    def compact(i, _):
        kv = lane + i * 16
        cv = hist[pl.ds(i * 16, 16)]
        m = (cv > 0) & (kv < nq)
        nc = cnt[2]
        offs = plsc.cumsum(jnp.where(m, 1, 0)) - 1 + nc
        plsc.store_scatter(nbl, [offs], kv, mask=m)
        cnt[2] = nc + plsc.all_reduce_population_count(m)[0]
        return 0

    lax.fori_loop(0, 256 // 16, compact, 0)
    nn = cnt[2]

    cnt[3] = 0

    def append_row(j, bref):
        pos = srt_p[pl.ds(j, 16)][0]
        ipos = ev[pl.ds(pos, 16)][0] & 127
        fc = cnt[3]
        for k2 in range(_D // 16):
            g = plsc.load_gather(
                bref, [lane + k2 * 16, lax.broadcast(ipos, (16,))]
            )
            fbuf[fc, pl.ds(k2 * 16, 16)] = g
        fpos[...] = jnp.where(
            lane == fc, lax.broadcast(pos, (16,)), fpos[...]
        )
        fc = fc + 1

        @pl.when(fc == _F)
        def _():
            pltpu.async_copy(fbuf, out.at[fpos], semf).wait()

        cnt[3] = jnp.where(fc == _F, 0, fc)
        return 0

    def process_block(k, bref):
        lo = st0[pl.ds(k, 16)][0]
        c = hist[pl.ds(k, 16)][0]
        lax.fori_loop(lo, lo + c, lambda j, _: append_row(j, bref), 0)

    sems = (sem0, sem1, sem2, sem3, sem4, sem5)

    def blkid(idx):
        return nbl[pl.ds(jnp.minimum(idx, 271), 16)][0]

    def issue(idx, b, sem):
        q = qlo + blkid(idx)
        pltpu.async_copy(tabT.at[:, pl.ds(q * 128, 128)], blk.at[b], sem)

    def wait(b, sem):
        pltpu.make_async_copy(
            tabT.at[:, pl.ds(0, 128)], blk.at[b], sem
        ).wait()

    for b in range(6):
        issue(b, b, sems[b])

    def hexa(t, _):
        for b in range(6):
            idx = t * 6 + b
            wait(b, sems[b])
            process_block(blkid(idx), blk.at[b])
            issue(idx + 6, b, sems[b])
        return 0

    nhex = lax.div(nn, 6) + 1
    lax.fori_loop(0, nhex, hexa, 0)
    for b in range(6):
        wait(b, sems[b])

    fc = cnt[3]
    dump = _B + wid * _F + lane
    fpos[...] = jnp.where(lane < fc, fpos[...], dump)
    pltpu.async_copy(fbuf, out.at[fpos], semf).wait()


def _tc_body(relT_ref, tailT_ref, r_ref, e_ref, rel_out_ref, tail_out_ref):
    r = r_ref[0, 0, :]
    e = e_ref[0, 0, :]
    iota_r = lax.broadcasted_iota(jnp.int32, (_R, 512), 0)
    oh_r = (iota_r == r[None, :]).astype(jnp.float32)
    rel_out_ref[...] = jnp.dot(
        relT_ref[...], oh_r, preferred_element_type=jnp.float32,
        precision=lax.Precision.HIGHEST
    )
    iota_t = lax.broadcasted_iota(jnp.int32, (_D, 512), 0)
    oh_t = (iota_t == (e[None, :] - _QTAIL)).astype(jnp.float32)
    tail_out_ref[...] = jnp.dot(
        tailT_ref[...], oh_t, preferred_element_type=jnp.float32,
        precision=lax.Precision.HIGHEST
    )


_tc_call = pl.pallas_call(
    _tc_body,
    grid=(_B // 512,),
    in_specs=[
        pl.BlockSpec((_D, _R), lambda j: (0, 0)),
        pl.BlockSpec((_D, _D), lambda j: (0, 0)),
        pl.BlockSpec((1, 1, 512), lambda j: (j, 0, 0)),
        pl.BlockSpec((1, 1, 512), lambda j: (j, 0, 0)),
    ],
    out_specs=[
        pl.BlockSpec((_D, 512), lambda j: (0, j)),
        pl.BlockSpec((_D, 512), lambda j: (0, j)),
    ],
    out_shape=[
        jax.ShapeDtypeStruct((_D, _B), jnp.float32),
        jax.ShapeDtypeStruct((_D, _B), jnp.float32),
    ],
)


def kernel(entity_table, relation_table, e, r):
    e = e.astype(jnp.int32)
    r = r.astype(jnp.int32)
    tabT = entity_table.T
    relT = relation_table.T
    tailT = lax.slice(tabT, (0, _QTAIL), (_D, _V))

    scratch = _ent_gather(tabT, e)
    relT_out, tailT_out = _tc_call(
        relT, tailT, r.reshape(_B // 512, 1, 512), e.reshape(_B // 512, 1, 512)
    )

    ent_sc = scratch[:_B, :_D]
    ent_emb = jnp.where((e >= _QTAIL)[:, None], tailT_out.T, ent_sc)
    rel_emb = relT_out.T
    return ent_emb, rel_emb


# speedup vs baseline: 1.0679x; 1.0013x over previous
import functools

import jax
import jax.numpy as jnp
from jax import lax
from jax.experimental import pallas as pl
from jax.experimental.pallas import tpu as pltpu
from jax.experimental.pallas import tpu_sc as plsc

_INFO = plsc.get_sparse_core_info()
_NC = _INFO.num_cores
_NS = _INFO.num_subcores
_NW = _NC * _NS

_B = 16384
_D = 64
_V = 1000000
_R = 1000
_NQ = 7812
_QTAIL = _NQ * 128
_QPER = _NQ // _NW
_QREM = _NQ % _NW
_F = 16
_SR = _B + _NW * _F

_MESH = plsc.VectorSubcoreMesh(core_axis_name="c", subcore_axis_name="s")


@functools.partial(
    pl.kernel,
    mesh=_MESH,
    out_type=jax.ShapeDtypeStruct((_SR, 128), jnp.float32),
    scratch_types=[
        pltpu.VMEM((_B + 16,), jnp.int32),
        pltpu.VMEM((_B + 16,), jnp.int32),
        pltpu.VMEM((_B + 16,), jnp.int32),
        pltpu.VMEM((288,), jnp.int32),
        pltpu.VMEM((288,), jnp.int32),
        pltpu.VMEM((288,), jnp.int32),
        pltpu.VMEM((288,), jnp.int32),
        pltpu.VMEM((6, _D, 128), jnp.float32),
        pltpu.VMEM((_F, 128), jnp.float32),
        pltpu.VMEM((16,), jnp.int32),
        pltpu.SMEM((8,), jnp.int32),
        pltpu.SemaphoreType.DMA,
        pltpu.SemaphoreType.DMA,
        pltpu.SemaphoreType.DMA,
        pltpu.SemaphoreType.DMA,
        pltpu.SemaphoreType.DMA,
        pltpu.SemaphoreType.DMA,
        pltpu.SemaphoreType.DMA,
    ],
    compiler_params=pltpu.CompilerParams(
        use_tc_tiling_on_sc=True, needs_layout_passes=False
    ),
)
def _ent_gather(tabT, e_hbm, out, ev, lst_p, srt_p, hist, st0, cur,
                nbl, blk, fbuf, fpos, cnt,
                sem0, sem1, sem2, sem3, sem4, sem5, semf):
    wid = lax.axis_index("s") * _NC + lax.axis_index("c")
    qlo = wid * _QPER + jnp.minimum(wid, _QREM)
    nq = jnp.where(wid < _QREM, _QPER + 1, _QPER)
    lane = lax.iota(jnp.int32, 16)
    zeros16 = jnp.zeros((16,), jnp.int32)

    pltpu.sync_copy(e_hbm, ev.at[pl.ds(0, _B)])

    cnt[0] = 0

    def filt(i, _):
        x = ev[pl.ds(i * 16, 16)]
        q = lax.shift_right_logical(x, 7)
        m = (q >= qlo) & (q < qlo + nq)
        offs = plsc.cumsum(jnp.where(m, 1, 0)) - 1 + cnt[0]
        plsc.store_scatter(lst_p, [offs], lane + i * 16, mask=m)
        cnt[0] = cnt[0] + plsc.all_reduce_population_count(m)[0]
        return 0

    lax.fori_loop(0, _B // 16, filt, 0)
    lcount = cnt[0]

    def zero18(i, _):
        hist[pl.ds(i * 16, 16)] = zeros16
        return 0

    lax.fori_loop(0, 288 // 16, zero18, 0)

    def count1(j, _):
        p_j = lst_p[pl.ds(j, 16)][0]
        e_j = ev[pl.ds(p_j, 16)][0]
        k = lax.shift_right_logical(e_j, 7) - qlo
        w = hist[pl.ds(k, 16)]
        hist[pl.ds(k, 16)] = jnp.where(lane == 0, w + 1, w)
        return 0

    lax.fori_loop(0, lcount, count1, 0)

    cnt[1] = 0

    def scan1(i, _):
        v = hist[pl.ds(i * 16, 16)]
        c = plsc.cumsum(v)
        run = cnt[1]
        st0[pl.ds(i * 16, 16)] = run + c - v
        cur[pl.ds(i * 16, 16)] = run + c - v
        cnt[1] = run + c[15]
        return 0

    lax.fori_loop(0, 288 // 16, scan1, 0)

    def place(j, _):
        p_j = lst_p[pl.ds(j, 16)][0]
        e_j = ev[pl.ds(p_j, 16)][0]
        k = lax.shift_right_logical(e_j, 7) - qlo
        o = cur[pl.ds(k, 16)][0]
        w2 = srt_p[pl.ds(o, 16)]
        srt_p[pl.ds(o, 16)] = jnp.where(
            lane == 0, lax.broadcast(p_j, (16,)), w2
        )
        wc = cur[pl.ds(k, 16)]
        cur[pl.ds(k, 16)] = jnp.where(lane == 0, wc + 1, wc)
        return 0

    lax.fori_loop(0, lcount, place, 0)

    def zeronbl(i, _):
        nbl[pl.ds(i * 16, 16)] = zeros16
        return 0

    lax.fori_loop(0, 288 // 16, zeronbl, 0)
    cnt[2] = 0

    def compact(i, _):
        kv = lane + i * 16
        cv = hist[pl.ds(i * 16, 16)]
        m = (cv > 0) & (kv < nq)
        nc = cnt[2]
        offs = plsc.cumsum(jnp.where(m, 1, 0)) - 1 + nc
        plsc.store_scatter(nbl, [offs], kv, mask=m)
        cnt[2] = nc + plsc.all_reduce_population_count(m)[0]
        return 0

    lax.fori_loop(0, 256 // 16, compact, 0)
    nn = cnt[2]

    cnt[3] = 0

    def append_row(j, bref):
        pos = srt_p[pl.ds(j, 16)][0]
        ipos = ev[pl.ds(pos, 16)][0] & 127
        fc = cnt[3]
        for k2 in range(_D // 16):
            g = plsc.load_gather(
                bref, [lane + k2 * 16, lax.broadcast(ipos, (16,))]
            )
            fbuf[fc, pl.ds(k2 * 16, 16)] = g
        fpos[...] = jnp.where(
            lane == fc, lax.broadcast(pos, (16,)), fpos[...]
        )
        fc = fc + 1

        @pl.when(fc == _F)
        def _():
            pltpu.async_copy(fbuf, out.at[fpos], semf).wait()

        cnt[3] = jnp.where(fc == _F, 0, fc)
        return 0

    def process_block(k, bref):
        lo = st0[pl.ds(k, 16)][0]
        c = hist[pl.ds(k, 16)][0]
        lax.fori_loop(lo, lo + c, lambda j, _: append_row(j, bref), 0)

    sems = (sem0, sem1, sem2, sem3, sem4, sem5)

    def blkid(idx):
        return nbl[pl.ds(jnp.minimum(idx, 271), 16)][0]

    def issue(idx, b, sem):
        q = qlo + blkid(idx)
        pltpu.async_copy(tabT.at[:, pl.ds(q * 128, 128)], blk.at[b], sem)

    def wait(b, sem):
        pltpu.make_async_copy(
            tabT.at[:, pl.ds(0, 128)], blk.at[b], sem
        ).wait()

    for b in range(6):
        issue(b, b, sems[b])

    def hexa(t, _):
        for b in range(6):
            idx = t * 6 + b
            wait(b, sems[b])
            process_block(blkid(idx), blk.at[b])
            issue(idx + 6, b, sems[b])
        return 0

    nhex = lax.div(nn, 6) + 1
    lax.fori_loop(0, nhex, hexa, 0)
    for b in range(6):
        wait(b, sems[b])

    fc = cnt[3]
    dump = _B + wid * _F + lane
    fpos[...] = jnp.where(lane < fc, fpos[...], dump)
    pltpu.async_copy(fbuf, out.at[fpos], semf).wait()


def _tc_body(relT_ref, tailT_ref, r_ref, e_ref, rel_out_ref, tail_out_ref):
    r = r_ref[0, 0, :]
    e = e_ref[0, 0, :]
    iota_r = lax.broadcasted_iota(jnp.int32, (_R, 512), 0)
    oh_r = (iota_r == r[None, :]).astype(jnp.float32)
    rel_out_ref[...] = jnp.dot(
        relT_ref[...], oh_r, preferred_element_type=jnp.float32,
        precision=lax.Precision.HIGHEST
    )
    iota_t = lax.broadcasted_iota(jnp.int32, (_D, 512), 0)
    oh_t = (iota_t == (e[None, :] - _QTAIL)).astype(jnp.float32)
    tail_out_ref[...] = jnp.dot(
        tailT_ref[...], oh_t, preferred_element_type=jnp.float32,
        precision=lax.Precision.HIGHEST
    )


_tc_call = pl.pallas_call(
    _tc_body,
    grid=(_B // 512,),
    in_specs=[
        pl.BlockSpec((_D, _R), lambda j: (0, 0)),
        pl.BlockSpec((_D, _D), lambda j: (0, 0)),
        pl.BlockSpec((1, 1, 512), lambda j: (j, 0, 0)),
        pl.BlockSpec((1, 1, 512), lambda j: (j, 0, 0)),
    ],
    out_specs=[
        pl.BlockSpec((_D, 512), lambda j: (0, j)),
        pl.BlockSpec((_D, 512), lambda j: (0, j)),
    ],
    out_shape=[
        jax.ShapeDtypeStruct((_D, _B), jnp.float32),
        jax.ShapeDtypeStruct((_D, _B), jnp.float32),
    ],
)


def kernel(entity_table, relation_table, e, r):
    e = e.astype(jnp.int32)
    r = r.astype(jnp.int32)
    tabT = entity_table.T
    relT = relation_table.T
    tailT = lax.slice(tabT, (0, _QTAIL), (_D, _V))

    scratch = _ent_gather(tabT, e)
    relT_out, tailT_out = _tc_call(
        relT, tailT, r.reshape(_B // 512, 1, 512), e.reshape(_B // 512, 1, 512)
    )

    ent_sc = scratch[:_B, :_D]
    ent_emb = jnp.where((e >= _QTAIL)[:, None], tailT_out.T, ent_sc)
    rel_emb = relT_out.T
    return ent_emb, rel_emb


# speedup vs baseline: 1.0690x; 1.0010x over previous
import functools

import jax
import jax.numpy as jnp
from jax import lax
from jax.experimental import pallas as pl
from jax.experimental.pallas import tpu as pltpu
from jax.experimental.pallas import tpu_sc as plsc

_INFO = plsc.get_sparse_core_info()
_NC = _INFO.num_cores
_NS = _INFO.num_subcores
_NW = _NC * _NS

_B = 16384
_D = 64
_V = 1000000
_R = 1000
_NQ = 7812
_QTAIL = _NQ * 128
_QPER = _NQ // _NW
_QREM = _NQ % _NW
_F = 16
_SR = _B + _NW * _F

_MESH = plsc.VectorSubcoreMesh(core_axis_name="c", subcore_axis_name="s")


@functools.partial(
    pl.kernel,
    mesh=_MESH,
    out_type=jax.ShapeDtypeStruct((_SR, 128), jnp.float32),
    scratch_types=[
        pltpu.VMEM((_B + 16,), jnp.int32),
        pltpu.VMEM((_B + 16,), jnp.int32),
        pltpu.VMEM((_B + 16,), jnp.int32),
        pltpu.VMEM((288,), jnp.int32),
        pltpu.VMEM((288,), jnp.int32),
        pltpu.VMEM((288,), jnp.int32),
        pltpu.VMEM((288,), jnp.int32),
        pltpu.VMEM((6, _D, 128), jnp.float32),
        pltpu.VMEM((_F, 128), jnp.float32),
        pltpu.VMEM((16,), jnp.int32),
        pltpu.SMEM((8,), jnp.int32),
        pltpu.SemaphoreType.DMA,
        pltpu.SemaphoreType.DMA,
        pltpu.SemaphoreType.DMA,
        pltpu.SemaphoreType.DMA,
        pltpu.SemaphoreType.DMA,
        pltpu.SemaphoreType.DMA,
        pltpu.SemaphoreType.DMA,
    ],
    compiler_params=pltpu.CompilerParams(
        use_tc_tiling_on_sc=True, needs_layout_passes=False
    ),
)
def _ent_gather(tabT, e_hbm, out, ev, lst_p, srt_p, hist, st0, cur,
                nbl, blk, fbuf, fpos, cnt,
                sem0, sem1, sem2, sem3, sem4, sem5, semf):
    wid = lax.axis_index("s") * _NC + lax.axis_index("c")
    qlo = wid * _QPER + jnp.minimum(wid, _QREM)
    nq = jnp.where(wid < _QREM, _QPER + 1, _QPER)
    lane = lax.iota(jnp.int32, 16)
    zeros16 = jnp.zeros((16,), jnp.int32)

    pltpu.sync_copy(e_hbm, ev.at[pl.ds(0, _B)])

    cnt[0] = 0

    def filt(i, _):
        lc = cnt[0]
        for u in range(4):
            base = i * 64 + u * 16
            x = ev[pl.ds(base, 16)]
            q = lax.shift_right_logical(x, 7)
            m = (q >= qlo) & (q < qlo + nq)
            offs = plsc.cumsum(jnp.where(m, 1, 0)) - 1 + lc
            plsc.store_scatter(lst_p, [offs], lane + base, mask=m)
            lc = lc + plsc.all_reduce_population_count(m)[0]
        cnt[0] = lc
        return 0

    lax.fori_loop(0, _B // 64, filt, 0)
    lcount = cnt[0]

    def zero18(i, _):
        hist[pl.ds(i * 16, 16)] = zeros16
        return 0

    lax.fori_loop(0, 288 // 16, zero18, 0)

    def count1(j, _):
        p_j = lst_p[pl.ds(j, 16)][0]
        e_j = ev[pl.ds(p_j, 16)][0]
        k = lax.shift_right_logical(e_j, 7) - qlo
        w = hist[pl.ds(k, 16)]
        hist[pl.ds(k, 16)] = jnp.where(lane == 0, w + 1, w)
        return 0

    lax.fori_loop(0, lcount, count1, 0)

    cnt[1] = 0

    def scan1(i, _):
        v = hist[pl.ds(i * 16, 16)]
        c = plsc.cumsum(v)
        run = cnt[1]
        st0[pl.ds(i * 16, 16)] = run + c - v
        cur[pl.ds(i * 16, 16)] = run + c - v
        cnt[1] = run + c[15]
        return 0

    lax.fori_loop(0, 288 // 16, scan1, 0)

    def place(j, _):
        p_j = lst_p[pl.ds(j, 16)][0]
        e_j = ev[pl.ds(p_j, 16)][0]
        k = lax.shift_right_logical(e_j, 7) - qlo
        o = cur[pl.ds(k, 16)][0]
        w2 = srt_p[pl.ds(o, 16)]
        srt_p[pl.ds(o, 16)] = jnp.where(
            lane == 0, lax.broadcast(p_j, (16,)), w2
        )
        wc = cur[pl.ds(k, 16)]
        cur[pl.ds(k, 16)] = jnp.where(lane == 0, wc + 1, wc)
        return 0

    lax.fori_loop(0, lcount, place, 0)

    def zeronbl(i, _):
        nbl[pl.ds(i * 16, 16)] = zeros16
        return 0

    lax.fori_loop(0, 288 // 16, zeronbl, 0)
    cnt[2] = 0

    def compact(i, _):
        kv = lane + i * 16
        cv = hist[pl.ds(i * 16, 16)]
        m = (cv > 0) & (kv < nq)
        nc = cnt[2]
        offs = plsc.cumsum(jnp.where(m, 1, 0)) - 1 + nc
        plsc.store_scatter(nbl, [offs], kv, mask=m)
        cnt[2] = nc + plsc.all_reduce_population_count(m)[0]
        return 0

    lax.fori_loop(0, 256 // 16, compact, 0)
    nn = cnt[2]

    cnt[3] = 0

    def append_row(j, bref):
        pos = srt_p[pl.ds(j, 16)][0]
        ipos = ev[pl.ds(pos, 16)][0] & 127
        fc = cnt[3]
        for k2 in range(_D // 16):
            g = plsc.load_gather(
                bref, [lane + k2 * 16, lax.broadcast(ipos, (16,))]
            )
            fbuf[fc, pl.ds(k2 * 16, 16)] = g
        fpos[...] = jnp.where(
            lane == fc, lax.broadcast(pos, (16,)), fpos[...]
        )
        fc = fc + 1

        @pl.when(fc == _F)
        def _():
            pltpu.async_copy(fbuf, out.at[fpos], semf).wait()

        cnt[3] = jnp.where(fc == _F, 0, fc)
        return 0

    def process_block(k, bref):
        lo = st0[pl.ds(k, 16)][0]
        c = hist[pl.ds(k, 16)][0]
        lax.fori_loop(lo, lo + c, lambda j, _: append_row(j, bref), 0)

    sems = (sem0, sem1, sem2, sem3, sem4, sem5)

    def blkid(idx):
        return nbl[pl.ds(jnp.minimum(idx, 271), 16)][0]

    def issue(idx, b, sem):
        q = qlo + blkid(idx)
        pltpu.async_copy(tabT.at[:, pl.ds(q * 128, 128)], blk.at[b], sem)

    def wait(b, sem):
        pltpu.make_async_copy(
            tabT.at[:, pl.ds(0, 128)], blk.at[b], sem
        ).wait()

    for b in range(6):
        issue(b, b, sems[b])

    def hexa(t, _):
        for b in range(6):
            idx = t * 6 + b
            wait(b, sems[b])
            process_block(blkid(idx), blk.at[b])
            issue(idx + 6, b, sems[b])
        return 0

    nhex = lax.div(nn, 6) + 1
    lax.fori_loop(0, nhex, hexa, 0)
    for b in range(6):
        wait(b, sems[b])

    fc = cnt[3]
    dump = _B + wid * _F + lane
    fpos[...] = jnp.where(lane < fc, fpos[...], dump)
    pltpu.async_copy(fbuf, out.at[fpos], semf).wait()


def _tc_body(relT_ref, tailT_ref, r_ref, e_ref, rel_out_ref, tail_out_ref):
    r = r_ref[0, 0, :]
    e = e_ref[0, 0, :]
    iota_r = lax.broadcasted_iota(jnp.int32, (_R, 512), 0)
    oh_r = (iota_r == r[None, :]).astype(jnp.float32)
    rel_out_ref[...] = jnp.dot(
        relT_ref[...], oh_r, preferred_element_type=jnp.float32,
        precision=lax.Precision.HIGHEST
    )
    iota_t = lax.broadcasted_iota(jnp.int32, (_D, 512), 0)
    oh_t = (iota_t == (e[None, :] - _QTAIL)).astype(jnp.float32)
    tail_out_ref[...] = jnp.dot(
        tailT_ref[...], oh_t, preferred_element_type=jnp.float32,
        precision=lax.Precision.HIGHEST
    )


_tc_call = pl.pallas_call(
    _tc_body,
    grid=(_B // 512,),
    in_specs=[
        pl.BlockSpec((_D, _R), lambda j: (0, 0)),
        pl.BlockSpec((_D, _D), lambda j: (0, 0)),
        pl.BlockSpec((1, 1, 512), lambda j: (j, 0, 0)),
        pl.BlockSpec((1, 1, 512), lambda j: (j, 0, 0)),
    ],
    out_specs=[
        pl.BlockSpec((_D, 512), lambda j: (0, j)),
        pl.BlockSpec((_D, 512), lambda j: (0, j)),
    ],
    out_shape=[
        jax.ShapeDtypeStruct((_D, _B), jnp.float32),
        jax.ShapeDtypeStruct((_D, _B), jnp.float32),
    ],
)


def kernel(entity_table, relation_table, e, r):
    e = e.astype(jnp.int32)
    r = r.astype(jnp.int32)
    tabT = entity_table.T
    relT = relation_table.T
    tailT = lax.slice(tabT, (0, _QTAIL), (_D, _V))

    scratch = _ent_gather(tabT, e)
    relT_out, tailT_out = _tc_call(
        relT, tailT, r.reshape(_B // 512, 1, 512), e.reshape(_B // 512, 1, 512)
    )

    ent_sc = scratch[:_B, :_D]
    ent_emb = jnp.where((e >= _QTAIL)[:, None], tailT_out.T, ent_sc)
    rel_emb = relT_out.T
    return ent_emb, rel_emb
